# baseline (speedup 1.0000x reference)
import functools

import jax
import jax.numpy as jnp
from jax import lax
from jax.experimental import pallas as pl
from jax.experimental.pallas import tpu as pltpu
from jax.experimental.pallas import tpu_sc as plsc

NC, NS = 2, 16
NW = NC * NS
CHUNK = 128
NBUF = 2
RSTAGE = 32


def _emb_body(table_hbm, idx_hbm, out_hbm, idx_raw, idx_st, rows_v, gsem, osem):
    nrows, seq = idx_hbm.shape
    rpw = nrows // NW
    bpw = rpw * seq
    cps = RSTAGE * seq // CHUNK
    nstage = rpw // RSTAGE
    wid = lax.axis_index("s") * NC + lax.axis_index("c")
    base = wid * bpw
    lanes = lax.iota(jnp.int32, 16)
    rmul = (1 << 20) // seq + 1

    def gstart(j, b):
        pltpu.async_copy(
            table_hbm.at[idx_st.at[j]],
            rows_v.at[b], gsem.at[b])

    def gwait(b):
        pltpu.make_async_copy(
            table_hbm.at[idx_st.at[0]],
            rows_v.at[b], gsem.at[b]).wait()

    def ostart(off, b):
        pltpu.async_copy(
            rows_v.at[b],
            out_hbm.at[pl.ds(off, CHUNK)], osem.at[b])

    def owait(b):
        pltpu.make_async_copy(
            rows_v.at[b],
            out_hbm.at[pl.ds(base, CHUNK)], osem.at[b]).wait()

    @pl.loop(0, nstage)
    def _(t):
        pltpu.sync_copy(
            idx_hbm.at[pl.ds(wid * rpw + t * RSTAGE, RSTAGE)], idx_raw)

        @pl.loop(0, cps)
        def _(j):
            for k in range(CHUNK // 16):
                s = j * CHUNK + k * 16 + lanes
                r = lax.shift_right_logical(s * rmul, 20)
                c = s - r * seq
                idx_st[j, pl.ds(k * 16, 16)] = plsc.load_gather(
                    idx_raw, [r, c])

        sbase = base + t * cps * CHUNK
        for b in range(NBUF):
            gstart(b, b)

        @pl.loop(0, cps // NBUF)
        def _(g):
            for b in range(NBUF):
                j = g * NBUF + b
                gwait(b)
                ostart(sbase + j * CHUNK, b)
                owait(b)
                nxt = j + NBUF

                @pl.when(nxt < cps)
                def _():
                    gstart(nxt, b)


def _make_kernel(n_rows, seq, vocab, d_pad):
    rpw = n_rows // NW
    cps = RSTAGE * seq // CHUNK
    mesh = plsc.VectorSubcoreMesh(
        core_axis_name="c", subcore_axis_name="s",
        num_cores=NC, num_subcores=NS)
    return pl.kernel(
        _emb_body,
        out_type=jax.ShapeDtypeStruct((n_rows * seq, d_pad), jnp.float32),
        mesh=mesh,
        scratch_types=[
            pltpu.VMEM((RSTAGE, seq), jnp.int32),
            pltpu.VMEM((cps, CHUNK), jnp.int32),
            pltpu.VMEM((NBUF, CHUNK, d_pad), jnp.float32),
            pltpu.SemaphoreType.DMA((NBUF,)),
            pltpu.SemaphoreType.DMA((NBUF,)),
        ],
        compiler_params=pltpu.CompilerParams(use_tc_tiling_on_sc=True, needs_layout_passes=False),
    )


def kernel(input_ids, table):
    bsz, seq = input_ids.shape
    n_idx = bsz * seq
    vocab, d = table.shape
    d_pad = (d + 127) // 128 * 128
    if d_pad != d:
        table = jnp.pad(table, ((0, 0), (0, d_pad - d)))
    ids = input_ids.astype(jnp.int32)
    out = _make_kernel(bsz, seq, vocab, d_pad)(table, ids)
    return out[:, :d].reshape(bsz, seq, d)
